# baseline (speedup 1.0000x reference)
import functools

import jax
import jax.numpy as jnp
from jax import lax
from jax.experimental import pallas as pl
from jax.experimental.pallas import tpu as pltpu
from jax.experimental.pallas import tpu_sc as plsc

_K = 2
_E = 8
_D = 1024
_T = 4096
_B = 256
_P = _T * _K + _E * _B
_NB = _P // _B
_TT = 512

_NW = 32
_TPW = _T // _NW
_CH = 32
_NCH = _TPW // _CH

_mesh = plsc.VectorSubcoreMesh(core_axis_name="c", subcore_axis_name="s")


def _router_body(x_ref, wg_ref, bg_ref, idx_ref, prob_ref):
    scores = jnp.dot(x_ref[...], wg_ref[...],
                     preferred_element_type=jnp.float32) + bg_ref[...]
    col = lax.broadcasted_iota(jnp.int32, scores.shape, 1)
    s1 = jnp.max(scores, axis=1, keepdims=True)
    i1 = jnp.min(jnp.where(scores == s1, col, _E), axis=1, keepdims=True)
    masked = jnp.where(col == i1, -jnp.inf, scores)
    s2 = jnp.max(masked, axis=1, keepdims=True)
    i2 = jnp.min(jnp.where(masked == s2, col, _E), axis=1, keepdims=True)
    e2 = jnp.exp(s2 - s1)
    denom = 1.0 + e2
    idx_ref[:, 0:1] = i1
    idx_ref[:, 1:2] = i2
    prob_ref[:, 0:1] = 1.0 / denom
    prob_ref[:, 1:2] = e2 / denom


def _router(x, wg, bg2):
    return pl.pallas_call(
        _router_body,
        grid=(_T // _TT,),
        in_specs=[
            pl.BlockSpec((_TT, _D), lambda t: (t, 0)),
            pl.BlockSpec((_D, _E), lambda t: (0, 0)),
            pl.BlockSpec((1, _E), lambda t: (0, 0)),
        ],
        out_specs=[
            pl.BlockSpec((_TT, _K), lambda t: (t, 0)),
            pl.BlockSpec((_TT, _K), lambda t: (t, 0)),
        ],
        out_shape=[
            jax.ShapeDtypeStruct((_T, _K), jnp.int32),
            jax.ShapeDtypeStruct((_T, _K), jnp.float32),
        ],
    )(x, wg, bg2)


def _dispatch_plan(idx):
    flat_e = idx.reshape(-1)
    onehot = (flat_e[:, None] == jnp.arange(_E)[None, :]).astype(jnp.int32)
    counts = jnp.sum(onehot, axis=0)
    padded = ((counts + _B - 1) // _B) * _B
    starts = jnp.concatenate(
        [jnp.zeros((1,), padded.dtype), jnp.cumsum(padded)[:-1]])
    ends = starts + padded
    csum = jnp.cumsum(onehot, axis=0) - onehot
    rank = jnp.sum(csum * onehot, axis=1)
    start_a = jnp.sum(starts[None, :] * onehot, axis=1)
    pos2 = (start_a + rank).astype(jnp.int32).reshape(_T, _K)
    beid = jnp.minimum(
        jnp.sum((jnp.arange(_NB)[:, None] * _B >= ends[None, :])
                .astype(jnp.int32), axis=1),
        _E - 1).astype(jnp.int32)
    return pos2[:, 0], pos2[:, 1], beid


@functools.partial(
    pl.kernel, mesh=_mesh,
    out_type=jax.ShapeDtypeStruct((_P, _D), jnp.float32),
    scratch_types=[
        pltpu.VMEM((_CH, _D), jnp.float32),
        pltpu.VMEM((_CH,), jnp.int32),
        pltpu.VMEM((_CH,), jnp.int32),
        pltpu.SemaphoreType.DMA,
    ],
)
def _sc_dispatch(x_hbm, p0_hbm, p1_hbm, xs_hbm, rows_v, i0_v, i1_v, sem):
    wid = lax.axis_index("s") * 2 + lax.axis_index("c")
    base = wid * _TPW
    for c in range(_NCH):
        off = base + c * _CH
        pltpu.sync_copy(p0_hbm.at[pl.ds(off, _CH)], i0_v)
        pltpu.sync_copy(p1_hbm.at[pl.ds(off, _CH)], i1_v)
        pltpu.sync_copy(x_hbm.at[pl.ds(off, _CH)], rows_v)
        cp0 = pltpu.async_copy(rows_v, xs_hbm.at[i0_v], sem)
        cp1 = pltpu.async_copy(rows_v, xs_hbm.at[i1_v], sem)
        cp0.wait()
        cp1.wait()


def _gmm_body(eid_ref, xs_ref, we_ref, be_ref, ys_ref):
    ys_ref[...] = jnp.dot(xs_ref[...].astype(jnp.bfloat16), we_ref[0],
                          preferred_element_type=jnp.float32) + be_ref[0]


def _grouped_matmul(block_eid, xs, we, be):
    grid_spec = pltpu.PrefetchScalarGridSpec(
        num_scalar_prefetch=1,
        grid=(_NB,),
        in_specs=[
            pl.BlockSpec((_B, _D), lambda b, eid: (b, 0)),
            pl.BlockSpec((1, _D, _D), lambda b, eid: (eid[b], 0, 0)),
            pl.BlockSpec((1, 1, _D), lambda b, eid: (eid[b], 0, 0)),
        ],
        out_specs=pl.BlockSpec((_B, _D), lambda b, eid: (b, 0)),
    )
    return pl.pallas_call(
        _gmm_body,
        grid_spec=grid_spec,
        out_shape=jax.ShapeDtypeStruct((_P, _D), jnp.float32),
    )(block_eid, xs, we, be)


@functools.partial(
    pl.kernel, mesh=_mesh,
    out_type=(jax.ShapeDtypeStruct((_T, _D), jnp.float32),
              jax.ShapeDtypeStruct((_T, _D), jnp.float32)),
    scratch_types=[
        pltpu.VMEM((_CH, _D), jnp.float32),
        pltpu.VMEM((_CH, _D), jnp.float32),
        pltpu.VMEM((_CH,), jnp.int32),
        pltpu.VMEM((_CH,), jnp.int32),
        pltpu.SemaphoreType.DMA,
    ],
)
def _sc_collect(ys_hbm, p0_hbm, p1_hbm, a_hbm, b_hbm,
                ra_v, rb_v, i0_v, i1_v, sem):
    wid = lax.axis_index("s") * 2 + lax.axis_index("c")
    base = wid * _TPW
    for c in range(_NCH):
        off = base + c * _CH
        pltpu.sync_copy(p0_hbm.at[pl.ds(off, _CH)], i0_v)
        pltpu.sync_copy(p1_hbm.at[pl.ds(off, _CH)], i1_v)
        ca = pltpu.async_copy(ys_hbm.at[i0_v], ra_v, sem)
        cb = pltpu.async_copy(ys_hbm.at[i1_v], rb_v, sem)
        ca.wait()
        cb.wait()
        pltpu.sync_copy(ra_v, a_hbm.at[pl.ds(off, _CH)])
        pltpu.sync_copy(rb_v, b_hbm.at[pl.ds(off, _CH)])


def _combine_body(a_ref, b_ref, p_ref, out_ref):
    out_ref[...] = (p_ref[:, 0:1] * a_ref[...] + p_ref[:, 1:2] * b_ref[...])


def _combine(a, b, probs):
    return pl.pallas_call(
        _combine_body,
        grid=(_T // _TT,),
        in_specs=[
            pl.BlockSpec((_TT, _D), lambda t: (t, 0)),
            pl.BlockSpec((_TT, _D), lambda t: (t, 0)),
            pl.BlockSpec((_TT, _K), lambda t: (t, 0)),
        ],
        out_specs=pl.BlockSpec((_TT, _D), lambda t: (t, 0)),
        out_shape=jax.ShapeDtypeStruct((_T, _D), jnp.float32),
    )(a, b, probs)


def kernel(inputs, Wg, bg, We, be):
    idx, probs = _router(inputs, Wg, bg.reshape(1, _E))
    pos0, pos1, block_eid = _dispatch_plan(idx)
    xs = _sc_dispatch(inputs, pos0, pos1)
    ys = _grouped_matmul(block_eid, xs, We.astype(jnp.bfloat16),
                         be.reshape(_E, 1, _D))
    a, b = _sc_collect(ys, pos0, pos1)
    out = _combine(a, b, probs)
    return (out, probs)


# speedup vs baseline: 1.0658x; 1.0658x over previous
import functools

import jax
import jax.numpy as jnp
from jax import lax
from jax.experimental import pallas as pl
from jax.experimental.pallas import tpu as pltpu
from jax.experimental.pallas import tpu_sc as plsc

_K = 2
_E = 8
_D = 1024
_T = 4096
_B = 256
_P = _T * _K + _E * _B
_NB = _P // _B
_TT = 512

_NW = 32
_TPW = _T // _NW
_CH = 32
_NCH = _TPW // _CH

@functools.cache
def _get_mesh():
    return plsc.VectorSubcoreMesh(core_axis_name="c", subcore_axis_name="s")


def _router_body(x_ref, wg_ref, bg_ref, idx_ref, prob_ref, rank_ref, cnt_ref,
                 carry_ref):
    t = pl.program_id(0)

    @pl.when(t == 0)
    def _():
        carry_ref[...] = jnp.zeros_like(carry_ref)

    scores = jnp.dot(x_ref[...], wg_ref[...],
                     preferred_element_type=jnp.float32) + bg_ref[...]
    col = lax.broadcasted_iota(jnp.int32, scores.shape, 1)
    s1 = jnp.max(scores, axis=1, keepdims=True)
    i1 = jnp.min(jnp.where(scores == s1, col, _E), axis=1, keepdims=True)
    masked = jnp.where(col == i1, -jnp.inf, scores)
    s2 = jnp.max(masked, axis=1, keepdims=True)
    i2 = jnp.min(jnp.where(masked == s2, col, _E), axis=1, keepdims=True)
    e2 = jnp.exp(s2 - s1)
    denom = 1.0 + e2
    idx_ref[:, 0:1] = i1
    idx_ref[:, 1:2] = i2
    prob_ref[:, 0:1] = 1.0 / denom
    prob_ref[:, 1:2] = e2 / denom
    oh0 = (col == i1).astype(jnp.float32)
    oh1 = (col == i2).astype(jnp.float32)
    rowsum = oh0 + oh1
    r_io = lax.broadcasted_iota(jnp.int32, (_TT, _TT), 0)
    c_io = lax.broadcasted_iota(jnp.int32, (_TT, _TT), 1)
    ltri = (r_io > c_io).astype(jnp.float32)
    prefix = jnp.dot(ltri, rowsum,
                     preferred_element_type=jnp.float32) + carry_ref[...]
    rank_ref[:, 0:1] = jnp.sum(prefix * oh0, axis=1,
                               keepdims=True).astype(jnp.int32)
    rank_ref[:, 1:2] = jnp.sum(prefix * oh1, axis=1,
                               keepdims=True).astype(jnp.int32)
    carry_new = carry_ref[...] + jnp.sum(rowsum, axis=0, keepdims=True)
    carry_ref[...] = carry_new
    cnt_ref[...] = carry_new


def _router(x, wg, bg2):
    return pl.pallas_call(
        _router_body,
        grid=(_T // _TT,),
        in_specs=[
            pl.BlockSpec((_TT, _D), lambda t: (t, 0)),
            pl.BlockSpec((_D, _E), lambda t: (0, 0)),
            pl.BlockSpec((1, _E), lambda t: (0, 0)),
        ],
        out_specs=[
            pl.BlockSpec((_TT, _K), lambda t: (t, 0)),
            pl.BlockSpec((_TT, _K), lambda t: (t, 0)),
            pl.BlockSpec((_TT, _K), lambda t: (t, 0)),
            pl.BlockSpec((1, _E), lambda t: (0, 0)),
        ],
        out_shape=[
            jax.ShapeDtypeStruct((_T, _K), jnp.int32),
            jax.ShapeDtypeStruct((_T, _K), jnp.float32),
            jax.ShapeDtypeStruct((_T, _K), jnp.int32),
            jax.ShapeDtypeStruct((1, _E), jnp.float32),
        ],
        scratch_shapes=[pltpu.VMEM((1, _E), jnp.float32)],
    )(x, wg, bg2)


def _dispatch_plan(idx, rank, counts_f):
    counts = counts_f.reshape(_E).astype(jnp.int32)
    padded = ((counts + _B - 1) // _B) * _B
    starts = jnp.concatenate(
        [jnp.zeros((1,), padded.dtype), jnp.cumsum(padded)[:-1]])
    ends = starts + padded
    oh = idx[..., None] == jnp.arange(_E)[None, None, :]
    pos2 = jnp.sum(jnp.where(oh, starts[None, None, :], 0), axis=2) + rank
    pos2 = pos2.astype(jnp.int32)
    beid = jnp.minimum(
        jnp.sum((jnp.arange(_NB)[:, None] * _B >= ends[None, :])
                .astype(jnp.int32), axis=1),
        _E - 1).astype(jnp.int32)
    return pos2[:, 0], pos2[:, 1], beid


def _sc_dispatch(x, pos0, pos1):
    @functools.partial(
        pl.kernel, mesh=_get_mesh(),
        out_type=jax.ShapeDtypeStruct((_P, _D), jnp.float32),
        scratch_types=[
            pltpu.VMEM((_CH, _D), jnp.float32),
            pltpu.VMEM((_CH,), jnp.int32),
            pltpu.VMEM((_CH,), jnp.int32),
            pltpu.SemaphoreType.DMA,
        ],
    )
    def k(x_hbm, p0_hbm, p1_hbm, xs_hbm, rows_v, i0_v, i1_v, sem):
        wid = lax.axis_index("s") * 2 + lax.axis_index("c")
        base = wid * _TPW
        for c in range(_NCH):
            off = base + c * _CH
            pltpu.sync_copy(p0_hbm.at[pl.ds(off, _CH)], i0_v)
            pltpu.sync_copy(p1_hbm.at[pl.ds(off, _CH)], i1_v)
            pltpu.sync_copy(x_hbm.at[pl.ds(off, _CH)], rows_v)
            cp0 = pltpu.async_copy(rows_v, xs_hbm.at[i0_v], sem)
            cp1 = pltpu.async_copy(rows_v, xs_hbm.at[i1_v], sem)
            cp0.wait()
            cp1.wait()

    return k(x, pos0, pos1)


def _gmm_body(eid_ref, xs_ref, we_ref, be_ref, ys_ref):
    ys_ref[...] = jnp.dot(xs_ref[...], we_ref[0],
                          preferred_element_type=jnp.float32) + be_ref[0]


def _grouped_matmul(block_eid, xs, we, be):
    grid_spec = pltpu.PrefetchScalarGridSpec(
        num_scalar_prefetch=1,
        grid=(_NB,),
        in_specs=[
            pl.BlockSpec((_B, _D), lambda b, eid: (b, 0)),
            pl.BlockSpec((1, _D, _D), lambda b, eid: (eid[b], 0, 0)),
            pl.BlockSpec((1, 1, _D), lambda b, eid: (eid[b], 0, 0)),
        ],
        out_specs=pl.BlockSpec((_B, _D), lambda b, eid: (b, 0)),
    )
    return pl.pallas_call(
        _gmm_body,
        grid_spec=grid_spec,
        out_shape=jax.ShapeDtypeStruct((_P, _D), jnp.float32),
    )(block_eid, xs, we, be)


def _sc_collect(ys, pos0, pos1):
    @functools.partial(
        pl.kernel, mesh=_get_mesh(),
        out_type=(jax.ShapeDtypeStruct((_T, _D), jnp.float32),
                  jax.ShapeDtypeStruct((_T, _D), jnp.float32)),
        scratch_types=[
            pltpu.VMEM((_CH, _D), jnp.float32),
            pltpu.VMEM((_CH, _D), jnp.float32),
            pltpu.VMEM((_CH,), jnp.int32),
            pltpu.VMEM((_CH,), jnp.int32),
            pltpu.SemaphoreType.DMA,
        ],
    )
    def k(ys_hbm, p0_hbm, p1_hbm, a_hbm, b_hbm,
          ra_v, rb_v, i0_v, i1_v, sem):
        wid = lax.axis_index("s") * 2 + lax.axis_index("c")
        base = wid * _TPW
        for c in range(_NCH):
            off = base + c * _CH
            pltpu.sync_copy(p0_hbm.at[pl.ds(off, _CH)], i0_v)
            pltpu.sync_copy(p1_hbm.at[pl.ds(off, _CH)], i1_v)
            ca = pltpu.async_copy(ys_hbm.at[i0_v], ra_v, sem)
            cb = pltpu.async_copy(ys_hbm.at[i1_v], rb_v, sem)
            ca.wait()
            cb.wait()
            pltpu.sync_copy(ra_v, a_hbm.at[pl.ds(off, _CH)])
            pltpu.sync_copy(rb_v, b_hbm.at[pl.ds(off, _CH)])

    return k(ys, pos0, pos1)


def _combine_body(a_ref, b_ref, p_ref, out_ref):
    out_ref[...] = (p_ref[:, 0:1] * a_ref[...] + p_ref[:, 1:2] * b_ref[...])


def _combine(a, b, probs):
    return pl.pallas_call(
        _combine_body,
        grid=(_T // _TT,),
        in_specs=[
            pl.BlockSpec((_TT, _D), lambda t: (t, 0)),
            pl.BlockSpec((_TT, _D), lambda t: (t, 0)),
            pl.BlockSpec((_TT, _K), lambda t: (t, 0)),
        ],
        out_specs=pl.BlockSpec((_TT, _D), lambda t: (t, 0)),
        out_shape=jax.ShapeDtypeStruct((_T, _D), jnp.float32),
    )(a, b, probs)


def kernel(inputs, Wg, bg, We, be):
    idx, probs, rank, counts_f = _router(inputs, Wg, bg.reshape(1, _E))
    pos0, pos1, block_eid = _dispatch_plan(idx, rank, counts_f)
    xs = _sc_dispatch(inputs, pos0, pos1)
    ys = _grouped_matmul(block_eid, xs, We, be.reshape(_E, 1, _D))
    a, b = _sc_collect(ys, pos0, pos1)
    out = _combine(a, b, probs)
    return (out, probs)


# speedup vs baseline: 1.1046x; 1.0364x over previous
import functools

import jax
import jax.numpy as jnp
from jax import lax
from jax.experimental import pallas as pl
from jax.experimental.pallas import tpu as pltpu
from jax.experimental.pallas import tpu_sc as plsc

_K = 2
_E = 8
_D = 1024
_T = 4096
_B = 256
_P = _T * _K + _E * _B
_NB = _P // _B
_TT = 512

_NW = 32
_TPW = _T // _NW
_CH = 32
_NCH = _TPW // _CH

@functools.cache
def _get_mesh():
    return plsc.VectorSubcoreMesh(core_axis_name="c", subcore_axis_name="s")


def _router_body(x_ref, wg_ref, bg_ref, idx_ref, prob_ref, rank_ref, cnt_ref,
                 carry_ref):
    t = pl.program_id(0)

    @pl.when(t == 0)
    def _():
        carry_ref[...] = jnp.zeros_like(carry_ref)

    scores = jnp.dot(x_ref[...], wg_ref[...],
                     preferred_element_type=jnp.float32) + bg_ref[...]
    col = lax.broadcasted_iota(jnp.int32, scores.shape, 1)
    s1 = jnp.max(scores, axis=1, keepdims=True)
    i1 = jnp.min(jnp.where(scores == s1, col, _E), axis=1, keepdims=True)
    masked = jnp.where(col == i1, -jnp.inf, scores)
    s2 = jnp.max(masked, axis=1, keepdims=True)
    i2 = jnp.min(jnp.where(masked == s2, col, _E), axis=1, keepdims=True)
    e2 = jnp.exp(s2 - s1)
    denom = 1.0 + e2
    idx_ref[:, 0:1] = i1
    idx_ref[:, 1:2] = i2
    prob_ref[:, 0:1] = 1.0 / denom
    prob_ref[:, 1:2] = e2 / denom
    oh0 = (col == i1).astype(jnp.float32)
    oh1 = (col == i2).astype(jnp.float32)
    rowsum = oh0 + oh1
    r_io = lax.broadcasted_iota(jnp.int32, (_TT, _TT), 0)
    c_io = lax.broadcasted_iota(jnp.int32, (_TT, _TT), 1)
    ltri = (r_io > c_io).astype(jnp.float32)
    prefix = jnp.dot(ltri, rowsum,
                     preferred_element_type=jnp.float32) + carry_ref[...]
    rank_ref[:, 0:1] = jnp.sum(prefix * oh0, axis=1,
                               keepdims=True).astype(jnp.int32)
    rank_ref[:, 1:2] = jnp.sum(prefix * oh1, axis=1,
                               keepdims=True).astype(jnp.int32)
    carry_new = carry_ref[...] + jnp.sum(rowsum, axis=0, keepdims=True)
    carry_ref[...] = carry_new
    cnt_ref[...] = carry_new


def _router(x, wg, bg2):
    return pl.pallas_call(
        _router_body,
        grid=(_T // _TT,),
        in_specs=[
            pl.BlockSpec((_TT, _D), lambda t: (t, 0)),
            pl.BlockSpec((_D, _E), lambda t: (0, 0)),
            pl.BlockSpec((1, _E), lambda t: (0, 0)),
        ],
        out_specs=[
            pl.BlockSpec((_TT, _K), lambda t: (t, 0)),
            pl.BlockSpec((_TT, _K), lambda t: (t, 0)),
            pl.BlockSpec((_TT, _K), lambda t: (t, 0)),
            pl.BlockSpec((1, _E), lambda t: (0, 0)),
        ],
        out_shape=[
            jax.ShapeDtypeStruct((_T, _K), jnp.int32),
            jax.ShapeDtypeStruct((_T, _K), jnp.float32),
            jax.ShapeDtypeStruct((_T, _K), jnp.int32),
            jax.ShapeDtypeStruct((1, _E), jnp.float32),
        ],
        scratch_shapes=[pltpu.VMEM((1, _E), jnp.float32)],
    )(x, wg, bg2)


def _dispatch_plan(idx, rank, counts_f):
    counts = counts_f.reshape(_E).astype(jnp.int32)
    padded = ((counts + _B - 1) // _B) * _B
    starts = jnp.concatenate(
        [jnp.zeros((1,), padded.dtype), jnp.cumsum(padded)[:-1]])
    ends = starts + padded
    oh = idx[..., None] == jnp.arange(_E)[None, None, :]
    pos2 = jnp.sum(jnp.where(oh, starts[None, None, :], 0), axis=2) + rank
    pos2 = pos2.astype(jnp.int32)
    beid = jnp.minimum(
        jnp.sum((jnp.arange(_NB)[:, None] * _B >= ends[None, :])
                .astype(jnp.int32), axis=1),
        _E - 1).astype(jnp.int32)
    return pos2[:, 0], pos2[:, 1], beid


def _sc_dispatch(x, pos0, pos1):
    @functools.partial(
        pl.kernel, mesh=_get_mesh(),
        out_type=jax.ShapeDtypeStruct((_P, _D), jnp.float32),
        scratch_types=[
            pltpu.VMEM((_CH, _D), jnp.float32),
            pltpu.VMEM((_CH,), jnp.int32),
            pltpu.VMEM((_CH,), jnp.int32),
            pltpu.SemaphoreType.DMA,
        ],
    )
    def k(x_hbm, p0_hbm, p1_hbm, xs_hbm, rows_v, i0_v, i1_v, sem):
        wid = lax.axis_index("s") * 2 + lax.axis_index("c")
        base = wid * _TPW
        for c in range(_NCH):
            off = base + c * _CH
            pltpu.sync_copy(p0_hbm.at[pl.ds(off, _CH)], i0_v)
            pltpu.sync_copy(p1_hbm.at[pl.ds(off, _CH)], i1_v)
            pltpu.sync_copy(x_hbm.at[pl.ds(off, _CH)], rows_v)
            cp0 = pltpu.async_copy(rows_v, xs_hbm.at[i0_v], sem)
            cp1 = pltpu.async_copy(rows_v, xs_hbm.at[i1_v], sem)
            cp0.wait()
            cp1.wait()

    return k(x, pos0, pos1)


def _gmm_body(eid_ref, xs_ref, we_ref, be_ref, ys_ref):
    ys_ref[...] = jnp.dot(xs_ref[...], we_ref[0],
                          preferred_element_type=jnp.float32) + be_ref[0]


def _grouped_matmul(block_eid, xs, we, be):
    grid_spec = pltpu.PrefetchScalarGridSpec(
        num_scalar_prefetch=1,
        grid=(_NB,),
        in_specs=[
            pl.BlockSpec((_B, _D), lambda b, eid: (b, 0)),
            pl.BlockSpec((1, _D, _D), lambda b, eid: (eid[b], 0, 0)),
            pl.BlockSpec((1, 1, _D), lambda b, eid: (eid[b], 0, 0)),
        ],
        out_specs=pl.BlockSpec((_B, _D), lambda b, eid: (b, 0)),
    )
    return pl.pallas_call(
        _gmm_body,
        grid_spec=grid_spec,
        out_shape=jax.ShapeDtypeStruct((_P, _D), jnp.float32),
    )(block_eid, xs, we, be)


_CC = 16
_NCC = _TPW // _CC
_L = 16


def _sc_collect_combine(ys, pos0, pos1, pb0, pb1):
    @functools.partial(
        pl.kernel, mesh=_get_mesh(),
        out_type=jax.ShapeDtypeStruct((_T, _D), jnp.float32),
        scratch_types=[
            pltpu.VMEM((_CC, _D), jnp.float32),
            pltpu.VMEM((_CC, _D), jnp.float32),
            pltpu.VMEM((_CC, _D), jnp.float32),
            pltpu.VMEM((_CC, _D), jnp.float32),
            pltpu.VMEM((_CC, _D), jnp.float32),
            pltpu.VMEM((_NCC, _CC), jnp.int32),
            pltpu.VMEM((_NCC, _CC), jnp.int32),
            pltpu.VMEM((_TPW, _L), jnp.float32),
            pltpu.VMEM((_TPW, _L), jnp.float32),
            pltpu.SemaphoreType.DMA,
            pltpu.SemaphoreType.DMA,
        ],
    )
    def k(ys_hbm, p0_hbm, p1_hbm, pb0_hbm, pb1_hbm, out_hbm,
          a0_v, a1_v, b0_v, b1_v, o_v, i0_v, i1_v, q0_v, q1_v, s0, s1):
        wid = lax.axis_index("s") * 2 + lax.axis_index("c")
        base = wid * _TPW
        a_bufs, b_bufs, sems = (a0_v, a1_v), (b0_v, b1_v), (s0, s1)
        pltpu.sync_copy(pb0_hbm.at[pl.ds(base, _TPW)], q0_v)
        pltpu.sync_copy(pb1_hbm.at[pl.ds(base, _TPW)], q1_v)
        for c in range(_NCC):
            pltpu.sync_copy(p0_hbm.at[pl.ds(base + c * _CC, _CC)],
                            i0_v.at[c])
            pltpu.sync_copy(p1_hbm.at[pl.ds(base + c * _CC, _CC)],
                            i1_v.at[c])

        def issue(c):
            s = sems[c % 2]
            ca = pltpu.async_copy(ys_hbm.at[i0_v.at[c]], a_bufs[c % 2], s)
            cb = pltpu.async_copy(ys_hbm.at[i1_v.at[c]], b_bufs[c % 2], s)
            return ca, cb

        pend = issue(0)
        for c in range(_NCC):
            nxt = issue(c + 1) if c + 1 < _NCC else None
            pend[0].wait()
            pend[1].wait()
            a_v, b_v = a_bufs[c % 2], b_bufs[c % 2]

            def tok_body(t, carry):
                p0s = q0_v[c * _CC + t, :]
                p1s = q1_v[c * _CC + t, :]
                for j in range(_D // _L):
                    sl = pl.ds(j * _L, _L)
                    o_v[t, sl] = a_v[t, sl] * p0s + b_v[t, sl] * p1s
                return carry

            lax.fori_loop(0, _CC, tok_body, 0)
            pltpu.sync_copy(o_v, out_hbm.at[pl.ds(base + c * _CC, _CC)])
            pend = nxt

    return k(ys, pos0, pos1, pb0, pb1)


def kernel(inputs, Wg, bg, We, be):
    idx, probs, rank, counts_f = _router(inputs, Wg, bg.reshape(1, _E))
    pos0, pos1, block_eid = _dispatch_plan(idx, rank, counts_f)
    xs = _sc_dispatch(inputs, pos0, pos1)
    ys = _grouped_matmul(block_eid, xs, We, be.reshape(_E, 1, _D))
    pb0 = jnp.repeat(probs[:, 0:1], _L, axis=1)
    pb1 = jnp.repeat(probs[:, 1:2], _L, axis=1)
    out = _sc_collect_combine(ys, pos0, pos1, pb0, pb1)
    return (out, probs)
